# baseline (speedup 1.0000x reference)
import jax
import jax.numpy as jnp
from jax.experimental import pallas as pl
from jax.experimental.pallas import tpu as pltpu

_B, _D, _E = 8192, 4096, 64
_BM = 512
_NB = _B // _BM
_PREC = jax.lax.Precision.HIGHEST


def _gating_kernel(x_ref, w1_ref, b1_ref, w2_ref, b2_ref, w3_ref, b3_ref,
                   w4_ref, b4_ref, out_ref, acc_ref):
    i = pl.program_id(0)
    nb = pl.num_programs(0)

    @pl.when(i == 0)
    def _init():
        acc_ref[0, 0] = 0.0

    x = x_ref[...]
    h = jnp.dot(x, w1_ref[...], preferred_element_type=jnp.float32,
                precision=_PREC) + b1_ref[...]
    h = jnp.maximum(h, 0.0)
    h = jnp.dot(h, w2_ref[...], preferred_element_type=jnp.float32,
                precision=_PREC) + b2_ref[...]
    h = jnp.where(h >= 0, h, 0.01 * h)
    h = jnp.dot(h, w3_ref[...], preferred_element_type=jnp.float32,
                precision=_PREC) + b3_ref[...]
    h = jnp.where(h >= 0, h, 0.01 * h)
    logits = jnp.dot(h, w4_ref[...], preferred_element_type=jnp.float32,
                     precision=_PREC) + b4_ref[...]

    lane = jax.lax.broadcasted_iota(jnp.int32, logits.shape, 1)
    v1 = jnp.max(logits, axis=1, keepdims=True)
    i1 = jnp.min(jnp.where(logits == v1, lane, _E), axis=1, keepdims=True)
    masked = jnp.where(lane == i1, -jnp.inf, logits)
    v2 = jnp.max(masked, axis=1, keepdims=True)
    i2 = jnp.min(jnp.where(masked == v2, lane, _E), axis=1, keepdims=True)

    acc_ref[0, 0] += jnp.sum(v1) + jnp.sum(v2)

    is_last = i == nb - 1
    total = acc_ref[0, 0]
    row = jax.lax.broadcasted_iota(jnp.int32, logits.shape, 0)
    sel1 = is_last & (row == 0) & (lane == i1)
    sel2 = is_last & (row == 0) & (lane == i2)
    out_ref[...] = jnp.where(sel1, v1 / total,
                             jnp.where(sel2, v2 / total, 0.0))


def kernel(x, W1, b1, W2, b2, W3, b3, W4, b4):
    w1t, w2t, w3t, w4t = W1.T, W2.T, W3.T, W4.T
    b1r = b1.reshape(1, -1)
    b2r = b2.reshape(1, -1)
    b3r = b3.reshape(1, -1)
    b4r = b4.reshape(1, -1)
    shift = lambda i: ((i + 1) % _NB, 0)
    pinned = lambda i: (0, 0)
    return pl.pallas_call(
        _gating_kernel,
        grid=(_NB,),
        in_specs=[
            pl.BlockSpec((_BM, _D), shift),
            pl.BlockSpec((_D, 128), pinned),
            pl.BlockSpec((1, 128), pinned),
            pl.BlockSpec((128, 256), pinned),
            pl.BlockSpec((1, 256), pinned),
            pl.BlockSpec((256, 128), pinned),
            pl.BlockSpec((1, 128), pinned),
            pl.BlockSpec((128, _E), pinned),
            pl.BlockSpec((1, _E), pinned),
        ],
        out_specs=pl.BlockSpec((_BM, _E), shift),
        out_shape=jax.ShapeDtypeStruct((_B, _E), jnp.float32),
        scratch_shapes=[pltpu.SMEM((1, 1), jnp.float32)],
    )(x, w1t, b1r, w2t, b2r, w3t, b3r, w4t, b4r)
